# baseline (speedup 1.0000x reference)
import functools

import jax
import jax.numpy as jnp
from jax import lax
from jax.experimental import pallas as pl
from jax.experimental.pallas import tpu as pltpu
from jax.experimental.pallas import tpu_sc as plsc

_N = 50000
_E = 800000
_C = 5
_F = 12
_PF = 8
_CF = _C * _F
_AW = _C * _PF
_YW = 32
_OW = 64
_BN = 5000
_K = 128
_NCORE = 2
_NSUB = 16
_CHT = 402
_EW = _CHT * _K
_ET = _NSUB * _EW
_HALF = _N // 2
_NPH = _HALF + 8
_ZR = 1568


def _pre_body(x_ref, wei_ref, wej_ref, be_ref, rexp_ref, wn1x_ref,
              wn1a_ref, bn1_ref, ptab_ref, ytab_ref, xpart_ref):
    xb = x_ref[...]
    ai = jnp.dot(xb, wei_ref[...], preferred_element_type=jnp.float32)
    aj = jnp.dot(xb, wej_ref[...], preferred_element_type=jnp.float32)
    aj = aj + be_ref[...]
    p = jnp.exp(ai - jnp.max(ai, axis=1, keepdims=True))
    q = jnp.exp(aj - jnp.max(aj, axis=1, keepdims=True))
    y = xb * jnp.dot(q, rexp_ref[...], preferred_element_type=jnp.float32)
    ym = jnp.dot(y, wn1a_ref[...], preferred_element_type=jnp.float32)
    xp = jnp.dot(xb, wn1x_ref[...], preferred_element_type=jnp.float32)
    z3 = jnp.zeros((xb.shape[0], 3), jnp.float32)
    ptab_ref[...] = jnp.concatenate([p, z3], axis=1)
    wa = lax.bitcast_convert_type(
        ym[:, :16].astype(jnp.bfloat16), jnp.uint16).astype(jnp.uint32)
    wb = lax.bitcast_convert_type(
        ym[:, 16:32].astype(jnp.bfloat16), jnp.uint16).astype(jnp.uint32)
    packed = lax.bitcast_convert_type(wa | (wb << jnp.uint32(16)),
                                      jnp.float32)
    ytab_ref[...] = jnp.concatenate([q, z3, packed, ym[:, 32:]], axis=1)
    xpart_ref[...] = xp + bn1_ref[...]


def _pre(x2, wei, wej, be1, rexp, wn1x, wn1a, bn1f):
    nblk = _N // _BN
    full = lambda shape: pl.BlockSpec(shape, lambda i: tuple(0 for _ in shape))
    return pl.pallas_call(
        _pre_body,
        grid=(nblk,),
        in_specs=[
            pl.BlockSpec((_BN, _CF), lambda i: (i, 0)),
            full((_CF, _C)), full((_CF, _C)), full((1, _C)),
            full((_C, _CF)), full((_CF, _AW)), full((_CF, _AW)),
            full((1, _AW)),
        ],
        out_specs=[
            pl.BlockSpec((_BN, 8), lambda i: (i, 0)),
            pl.BlockSpec((_BN, _YW), lambda i: (i, 0)),
            pl.BlockSpec((_BN, _AW), lambda i: (i, 0)),
        ],
        out_shape=[
            jax.ShapeDtypeStruct((_N, 8), jnp.float32),
            jax.ShapeDtypeStruct((_N, _YW), jnp.float32),
            jax.ShapeDtypeStruct((_N, _AW), jnp.float32),
        ],
    )(x2, wei, wej, be1, rexp, wn1x, wn1a, bn1f)


def _cvec(c):
    return jnp.full((16,), c, jnp.int32)


def _edge_body(epk_hbm, ptab_hbm, ytab_hbm, z_hbm, acc_hbm,
               idxb0, idxb1, pbuf0, pbuf1,
               ybuf0, ybuf1, obuf0, obuf1,
               dl0, dl1, accsh,
               si0, si1, sy0, sy1, sp0, sp1, ss0, ss1):
    cid = lax.axis_index("c")
    sid = lax.axis_index("s")
    idxb = (idxb0, idxb1)
    pbuf = (pbuf0, pbuf1)
    ybuf = (ybuf0, ybuf1)
    obuf = (obuf0, obuf1)
    dl = (dl0, dl1)
    si = (si0, si1)
    sy = (sy0, sy1)
    sp = (sp0, sp1)
    ss = (ss0, ss1)

    @pl.when(sid < _NSUB - 1)
    def _():
        pltpu.sync_copy(z_hbm, accsh.at[pl.ds(sid * _ZR, _ZR)])

    @pl.when(sid == _NSUB - 1)
    def _():
        rem = _NPH - (_NSUB - 1) * _ZR
        pltpu.sync_copy(z_hbm.at[pl.ds(0, rem)],
                        accsh.at[pl.ds((_NSUB - 1) * _ZR, rem)])

    plsc.subcore_barrier()

    lo = cid * _HALF
    cbase = sid * _CHT

    def issue_idx(i, b):
        pltpu.async_copy(epk_hbm.at[cbase + i], idxb[b], si[b])

    def wait_idx(i, b):
        pltpu.make_async_copy(epk_hbm.at[cbase + i], idxb[b], si[b]).wait()

    def issue_gathers(b):
        pltpu.async_copy(ytab_hbm.at[idxb[b].at[0]], ybuf[b], sy[b])
        pltpu.async_copy(ptab_hbm.at[idxb[b].at[1]], pbuf[b], sp[b])

    def wait_gathers(b):
        pltpu.make_async_copy(ytab_hbm.at[idxb[b].at[0]], ybuf[b],
                              sy[b]).wait()
        pltpu.make_async_copy(ptab_hbm.at[idxb[b].at[1]], pbuf[b],
                              sp[b]).wait()

    def issue_scatter(b):
        pltpu.async_copy(obuf[b], accsh.at[dl[b]], ss[b], add=True)

    def wait_scatter(b):
        pltpu.make_async_copy(obuf[b], accsh.at[dl[b]], ss[b]).wait()

    def compute(b):
        def group_body(g, c2):
            d = idxb[b][1, pl.ds(g * 16, 16)]
            inr = (d >= lo) & (d < lo + _HALF)
            dloc = jnp.where(inr, d - lo, _HALF)
            dl[b][pl.ds(g * 16, 16)] = dloc
            return c2

        lax.fori_loop(0, _K // 16, group_body, 0)

        def edge_body(e4, c2):
            for u in range(4):
                e = e4 * 4 + u
                q = ybuf[b][e, pl.ds(0, 16)]
                p = pbuf[b][e, pl.ds(0, 16)]
                zv = jnp.full((16,), jnp.sum(p * q), jnp.float32)
                rv = 1.0 / zv
                obuf[b][e, pl.ds(0, 16)] = q * rv
                w1 = ybuf[b][e, pl.ds(8, 16)]
                a1, b1 = plsc.unpack(plsc.bitcast(w1, jnp.bfloat16),
                                     format=plsc.PackFormat.INTERLEAVED)
                obuf[b][e, pl.ds(16, 16)] = a1 * rv
                obuf[b][e, pl.ds(32, 16)] = b1 * rv
                v3 = ybuf[b][e, pl.ds(16, 16)]
                obuf[b][e, pl.ds(48, 16)] = v3 * rv
            return c2

        lax.fori_loop(0, _K // 4, edge_body, 0)

    issue_idx(0, 0)
    issue_idx(1, 1)
    wait_idx(0, 0)
    issue_gathers(0)

    def outer(i2, carry):
        for b in range(2):
            i = i2 * 2 + b
            wait_gathers(b)

            @pl.when(i + 1 < _CHT)
            def _():
                wait_idx(i + 1, 1 - b)
                issue_gathers(1 - b)

            @pl.when(i2 >= 1)
            def _():
                wait_scatter(b)

            compute(b)
            issue_scatter(b)

            @pl.when(i + 2 < _CHT)
            def _():
                issue_idx(i + 2, b)
        return carry

    lax.fori_loop(0, _CHT // 2, outer, 0)
    wait_scatter(0)
    wait_scatter(1)

    plsc.subcore_barrier()

    @pl.when(sid < _NSUB - 1)
    def _():
        pltpu.sync_copy(accsh.at[pl.ds(sid * _ZR, _ZR)],
                        acc_hbm.at[cid, pl.ds(sid * _ZR, _ZR)])

    @pl.when(sid == _NSUB - 1)
    def _():
        rem = _HALF - (_NSUB - 1) * _ZR
        pltpu.sync_copy(accsh.at[pl.ds((_NSUB - 1) * _ZR, rem)],
                        acc_hbm.at[cid, pl.ds((_NSUB - 1) * _ZR, rem)])


def _edge(epk, ptab_p, ytab, zrows):
    mesh = plsc.VectorSubcoreMesh(core_axis_name="c", subcore_axis_name="s",
                                  num_cores=_NCORE, num_subcores=_NSUB)
    k = functools.partial(
        pl.kernel,
        out_type=jax.ShapeDtypeStruct((_NCORE, _HALF, _OW), jnp.float32),
        mesh=mesh,
        scratch_types=(
            [pltpu.VMEM((2, _K), jnp.int32)] * 2
            + [pltpu.VMEM((_K, 16), jnp.float32)] * 2
            + [pltpu.VMEM((_K, _YW), jnp.float32)] * 2
            + [pltpu.VMEM((_K, _OW), jnp.float32)] * 2
            + [pltpu.VMEM((_K,), jnp.int32)] * 2
            + [pltpu.VMEM_SHARED((_NPH, _OW), jnp.float32)]
            + [pltpu.SemaphoreType.DMA] * 8
        ),
        compiler_params=pltpu.CompilerParams(needs_layout_passes=False,
                                             use_tc_tiling_on_sc=False,
                                             internal_scratch_in_bytes=0),
    )(_edge_body)
    return k(epk, ptab_p, ytab, zrows)


def _post_body(ptab_ref, xpart_ref, acc_ref, r8_ref, wn2_ref, bn2_ref,
               out_ref):
    ab = acc_ref[...]
    g = jnp.concatenate([ab[:, 16:48], ab[:, 56:64]], axis=1)
    p5 = ptab_ref[...][:, :_C]
    pe = jnp.dot(p5, r8_ref[...], preferred_element_type=jnp.float32)
    h = jnp.tanh(xpart_ref[...] + pe * g)
    o = jnp.dot(h, wn2_ref[...], preferred_element_type=jnp.float32)
    out_ref[...] = jnp.tanh(o + bn2_ref[...])


def _post(ptab, xpart, acc, r8, wn2f, bn2f):
    nblk = _N // _BN
    full = lambda shape: pl.BlockSpec(shape, lambda i: tuple(0 for _ in shape))
    return pl.pallas_call(
        _post_body,
        grid=(nblk,),
        in_specs=[
            pl.BlockSpec((_BN, 8), lambda i: (i, 0)),
            pl.BlockSpec((_BN, _AW), lambda i: (i, 0)),
            pl.BlockSpec((_BN, _OW), lambda i: (i, 0)),
            full((_C, _AW)), full((_AW, _AW)), full((1, _AW)),
        ],
        out_specs=pl.BlockSpec((_BN, _AW), lambda i: (i, 0)),
        out_shape=jax.ShapeDtypeStruct((_N, _AW), jnp.float32),
    )(ptab, xpart, acc, r8, wn2f, bn2f)


def _plane_opt(x, edge_index, We, be, Wn1, bn1, Wn2, bn2):
    eye = jnp.eye(_C, dtype=jnp.float32)
    x2 = x.reshape(_N, _CF)
    wei = jnp.einsum('cf,ck->cfk', We[:, 0, :_F], eye).reshape(_CF, _C)
    wej = jnp.einsum('cf,ck->cfk', We[:, 0, _F:], eye).reshape(_CF, _C)
    be1 = be.reshape(1, _C)
    rexp = jnp.repeat(eye, _F, axis=1)
    wn1x = jnp.einsum('cof,ck->cfko', Wn1[:, :, :_F], eye).reshape(_CF, _AW)
    wn1a = jnp.einsum('cof,ck->cfko', Wn1[:, :, _F:], eye).reshape(_CF, _AW)
    bn1f = bn1.reshape(1, _AW)
    r8 = jnp.repeat(eye, _PF, axis=1)
    wn2f = jnp.einsum('coi,ck->ciko', Wn2, eye).reshape(_AW, _AW)
    bn2f = bn2.reshape(1, _AW)

    ptab, ytab, xpart = _pre(x2, wei, wej, be1, rexp, wn1x, wn1a, bn1f)

    npad = _ET - _E
    src = jnp.concatenate([edge_index[0], jnp.zeros((npad,), jnp.int32)])
    dst = jnp.concatenate([edge_index[1], jnp.full((npad,), _N, jnp.int32)])
    epk = jnp.stack([src.reshape(-1, _K), dst.reshape(-1, _K)], axis=1)
    ptab16 = jnp.concatenate([ptab, jnp.zeros((_N, 8), jnp.float32)], axis=1)
    drow = jnp.concatenate([jnp.ones((8, _C), jnp.float32),
                            jnp.zeros((8, 11), jnp.float32)], axis=1)
    ptab_p = jnp.concatenate([ptab16, drow], axis=0)
    zrows = jnp.zeros((_ZR, _OW), jnp.float32)

    acc = _edge(epk, ptab_p, ytab, zrows)

    out = _post(ptab, xpart, acc.reshape(_N, _OW), r8, wn2f, bn2f)
    return out.reshape(_N, _C, _PF)


def kernel(x_u, edge_index_u, We_u, be_u, Wn1_u, bn1_u, Wn2_u, bn2_u,
           x_v, edge_index_v, We_v, be_v, Wn1_v, bn1_v, Wn2_v, bn2_v,
           x_y, edge_index_y, We_y, be_y, Wn1_y, bn1_y, Wn2_y, bn2_y):
    ou = _plane_opt(x_u, edge_index_u, We_u, be_u, Wn1_u, bn1_u, Wn2_u, bn2_u)
    ov = _plane_opt(x_v, edge_index_v, We_v, be_v, Wn1_v, bn1_v, Wn2_v, bn2_v)
    oy = _plane_opt(x_y, edge_index_y, We_y, be_y, Wn1_y, bn1_y, Wn2_y, bn2_y)
    return (ou, ov, oy)


# speedup vs baseline: 1.2260x; 1.2260x over previous
import functools

import jax
import jax.numpy as jnp
from jax import lax
from jax.experimental import pallas as pl
from jax.experimental.pallas import tpu as pltpu
from jax.experimental.pallas import tpu_sc as plsc

_N = 50000
_E = 800000
_C = 5
_F = 12
_PF = 8
_CF = _C * _F
_AW = _C * _PF
_YW = 48
_BN = 5000
_K = 128
_NCORE = 2
_NSUB = 16
_CHT = 400
_EW = _CHT * _K
_ET = _NSUB * _EW
_HALF = _N // 2
_NPH = _HALF + 8
_ZR = 1568


def _pre_body(x_ref, wei_ref, wej_ref, be_ref, rexp_ref, wn1x_ref,
              wn1a_ref, bn1_ref, ptab_ref, ytab_ref, xpart_ref):
    xb = x_ref[...]
    ai = jnp.dot(xb, wei_ref[...], preferred_element_type=jnp.float32)
    aj = jnp.dot(xb, wej_ref[...], preferred_element_type=jnp.float32)
    aj = aj + be_ref[...]
    p = jnp.exp(ai - jnp.max(ai, axis=1, keepdims=True))
    q = jnp.exp(aj - jnp.max(aj, axis=1, keepdims=True))
    y = xb * jnp.dot(q, rexp_ref[...], preferred_element_type=jnp.float32)
    ym = jnp.dot(y, wn1a_ref[...], preferred_element_type=jnp.float32)
    xp = jnp.dot(xb, wn1x_ref[...], preferred_element_type=jnp.float32)
    z3 = jnp.zeros((xb.shape[0], 3), jnp.float32)
    ptab_ref[...] = jnp.concatenate([p, z3], axis=1)
    ytab_ref[...] = jnp.concatenate([q, z3, ym], axis=1)
    xpart_ref[...] = xp + bn1_ref[...]


def _pre(x2, wei, wej, be1, rexp, wn1x, wn1a, bn1f):
    nblk = _N // _BN
    full = lambda shape: pl.BlockSpec(shape, lambda i: tuple(0 for _ in shape))
    return pl.pallas_call(
        _pre_body,
        grid=(nblk,),
        in_specs=[
            pl.BlockSpec((_BN, _CF), lambda i: (i, 0)),
            full((_CF, _C)), full((_CF, _C)), full((1, _C)),
            full((_C, _CF)), full((_CF, _AW)), full((_CF, _AW)),
            full((1, _AW)),
        ],
        out_specs=[
            pl.BlockSpec((_BN, 8), lambda i: (i, 0)),
            pl.BlockSpec((_BN, _YW), lambda i: (i, 0)),
            pl.BlockSpec((_BN, _AW), lambda i: (i, 0)),
        ],
        out_shape=[
            jax.ShapeDtypeStruct((_N, 8), jnp.float32),
            jax.ShapeDtypeStruct((_N, _YW), jnp.float32),
            jax.ShapeDtypeStruct((_N, _AW), jnp.float32),
        ],
    )(x2, wei, wej, be1, rexp, wn1x, wn1a, bn1f)


def _cvec(c):
    return jnp.full((16,), c, jnp.int32)


def _edge_body(epk_hbm, ptab_hbm, ytab_hbm, z_hbm, acc_hbm,
               idxb0, idxb1, pbuf0, pbuf1,
               ybuf0, ybuf1, obuf0, obuf1,
               dl0, dl1, accsh,
               si0, si1, sy0, sy1, sp0, sp1, ss0, ss1):
    cid = lax.axis_index("c")
    sid = lax.axis_index("s")
    idxb = (idxb0, idxb1)
    pbuf = (pbuf0, pbuf1)
    ybuf = (ybuf0, ybuf1)
    obuf = (obuf0, obuf1)
    dl = (dl0, dl1)
    si = (si0, si1)
    sy = (sy0, sy1)
    sp = (sp0, sp1)
    ss = (ss0, ss1)

    @pl.when(sid < _NSUB - 1)
    def _():
        pltpu.sync_copy(z_hbm, accsh.at[pl.ds(sid * _ZR, _ZR)])

    @pl.when(sid == _NSUB - 1)
    def _():
        rem = _NPH - (_NSUB - 1) * _ZR
        pltpu.sync_copy(z_hbm.at[pl.ds(0, rem)],
                        accsh.at[pl.ds((_NSUB - 1) * _ZR, rem)])

    plsc.subcore_barrier()

    lo = cid * _HALF
    cbase = sid * _CHT

    def issue_idx(i, b):
        pltpu.async_copy(epk_hbm.at[cbase + i], idxb[b], si[b])

    def wait_idx(i, b):
        pltpu.make_async_copy(epk_hbm.at[cbase + i], idxb[b], si[b]).wait()

    def issue_gathers(b):
        pltpu.async_copy(ytab_hbm.at[idxb[b].at[0]], ybuf[b], sy[b])
        pltpu.async_copy(ptab_hbm.at[idxb[b].at[1]], pbuf[b], sp[b])

    def wait_gathers(b):
        pltpu.make_async_copy(ytab_hbm.at[idxb[b].at[0]], ybuf[b],
                              sy[b]).wait()
        pltpu.make_async_copy(ptab_hbm.at[idxb[b].at[1]], pbuf[b],
                              sp[b]).wait()

    def issue_scatter(b):
        pltpu.async_copy(obuf[b], accsh.at[dl[b]], ss[b], add=True)

    def wait_scatter(b):
        pltpu.make_async_copy(obuf[b], accsh.at[dl[b]], ss[b]).wait()

    def compute(b):
        def group_body(g, c2):
            d = idxb[b][1, pl.ds(g * 16, 16)]
            inr = (d >= lo) & (d < lo + _HALF)
            dloc = jnp.where(inr, d - lo, _HALF)
            dl[b][pl.ds(g * 16, 16)] = dloc
            return c2

        lax.fori_loop(0, _K // 16, group_body, 0)

        def edge_body(e4, c2):
            for u in range(4):
                e = e4 * 4 + u
                q = ybuf[b][e, pl.ds(0, 16)]
                p = pbuf[b][e, pl.ds(0, 16)]
                zv = jnp.full((16,), jnp.sum(p * q), jnp.float32)
                rv = 1.0 / zv
                obuf[b][e, pl.ds(0, 16)] = q * rv
                for blk in (1, 2):
                    v = ybuf[b][e, pl.ds(16 * blk, 16)]
                    obuf[b][e, pl.ds(16 * blk, 16)] = v * rv
            return c2

        lax.fori_loop(0, _K // 4, edge_body, 0)

    issue_idx(0, 0)
    issue_idx(1, 1)
    wait_idx(0, 0)
    issue_gathers(0)

    def outer(i2, carry):
        for b in range(2):
            i = i2 * 2 + b
            wait_gathers(b)

            @pl.when(i + 1 < _CHT)
            def _():
                wait_idx(i + 1, 1 - b)
                issue_gathers(1 - b)

            @pl.when(i2 >= 1)
            def _():
                wait_scatter(b)

            compute(b)
            issue_scatter(b)

            @pl.when(i + 2 < _CHT)
            def _():
                issue_idx(i + 2, b)
        return carry

    lax.fori_loop(0, _CHT // 2, outer, 0)
    wait_scatter(0)
    wait_scatter(1)

    plsc.subcore_barrier()

    @pl.when(sid < _NSUB - 1)
    def _():
        pltpu.sync_copy(accsh.at[pl.ds(sid * _ZR, _ZR)],
                        acc_hbm.at[cid, pl.ds(sid * _ZR, _ZR)])

    @pl.when(sid == _NSUB - 1)
    def _():
        rem = _HALF - (_NSUB - 1) * _ZR
        pltpu.sync_copy(accsh.at[pl.ds((_NSUB - 1) * _ZR, rem)],
                        acc_hbm.at[cid, pl.ds((_NSUB - 1) * _ZR, rem)])


def _edge(epk, ptab_p, ytab, zrows):
    mesh = plsc.VectorSubcoreMesh(core_axis_name="c", subcore_axis_name="s",
                                  num_cores=_NCORE, num_subcores=_NSUB)
    k = functools.partial(
        pl.kernel,
        out_type=jax.ShapeDtypeStruct((_NCORE, _HALF, _YW), jnp.float32),
        mesh=mesh,
        scratch_types=(
            [pltpu.VMEM((2, _K), jnp.int32)] * 2
            + [pltpu.VMEM((_K, 16), jnp.float32)] * 2
            + [pltpu.VMEM((_K, _YW), jnp.float32)] * 2
            + [pltpu.VMEM((_K, _YW), jnp.float32)] * 2
            + [pltpu.VMEM((_K,), jnp.int32)] * 2
            + [pltpu.VMEM_SHARED((_NPH, _YW), jnp.float32)]
            + [pltpu.SemaphoreType.DMA] * 8
        ),
        compiler_params=pltpu.CompilerParams(needs_layout_passes=False,
                                             use_tc_tiling_on_sc=False,
                                             internal_scratch_in_bytes=0),
    )(_edge_body)
    return k(epk, ptab_p, ytab, zrows)


def _post_body(ptab_ref, xpart_ref, acc_ref, r8_ref, wn2_ref, bn2_ref,
               out_ref):
    g = acc_ref[...][:, 8:]
    p5 = ptab_ref[...][:, :_C]
    pe = jnp.dot(p5, r8_ref[...], preferred_element_type=jnp.float32)
    h = jnp.tanh(xpart_ref[...] + pe * g)
    o = jnp.dot(h, wn2_ref[...], preferred_element_type=jnp.float32)
    out_ref[...] = jnp.tanh(o + bn2_ref[...])


def _post(ptab, xpart, acc, r8, wn2f, bn2f):
    nblk = _N // _BN
    full = lambda shape: pl.BlockSpec(shape, lambda i: tuple(0 for _ in shape))
    return pl.pallas_call(
        _post_body,
        grid=(nblk,),
        in_specs=[
            pl.BlockSpec((_BN, 8), lambda i: (i, 0)),
            pl.BlockSpec((_BN, _AW), lambda i: (i, 0)),
            pl.BlockSpec((_BN, _YW), lambda i: (i, 0)),
            full((_C, _AW)), full((_AW, _AW)), full((1, _AW)),
        ],
        out_specs=pl.BlockSpec((_BN, _AW), lambda i: (i, 0)),
        out_shape=jax.ShapeDtypeStruct((_N, _AW), jnp.float32),
    )(ptab, xpart, acc, r8, wn2f, bn2f)


def _plane_opt(x, edge_index, We, be, Wn1, bn1, Wn2, bn2):
    eye = jnp.eye(_C, dtype=jnp.float32)
    x2 = x.reshape(_N, _CF)
    wei = jnp.einsum('cf,ck->cfk', We[:, 0, :_F], eye).reshape(_CF, _C)
    wej = jnp.einsum('cf,ck->cfk', We[:, 0, _F:], eye).reshape(_CF, _C)
    be1 = be.reshape(1, _C)
    rexp = jnp.repeat(eye, _F, axis=1)
    wn1x = jnp.einsum('cof,ck->cfko', Wn1[:, :, :_F], eye).reshape(_CF, _AW)
    wn1a = jnp.einsum('cof,ck->cfko', Wn1[:, :, _F:], eye).reshape(_CF, _AW)
    bn1f = bn1.reshape(1, _AW)
    r8 = jnp.repeat(eye, _PF, axis=1)
    wn2f = jnp.einsum('coi,ck->ciko', Wn2, eye).reshape(_AW, _AW)
    bn2f = bn2.reshape(1, _AW)

    ptab, ytab, xpart = _pre(x2, wei, wej, be1, rexp, wn1x, wn1a, bn1f)

    npad = _ET - _E
    src = jnp.concatenate([edge_index[0], jnp.zeros((npad,), jnp.int32)])
    dst = jnp.concatenate([edge_index[1], jnp.full((npad,), _N, jnp.int32)])
    epk = jnp.stack([src.reshape(-1, _K), dst.reshape(-1, _K)], axis=1)
    ptab16 = jnp.concatenate([ptab, jnp.zeros((_N, 8), jnp.float32)], axis=1)
    drow = jnp.concatenate([jnp.ones((8, _C), jnp.float32),
                            jnp.zeros((8, 11), jnp.float32)], axis=1)
    ptab_p = jnp.concatenate([ptab16, drow], axis=0)
    zrows = jnp.zeros((_ZR, _YW), jnp.float32)

    acc = _edge(epk, ptab_p, ytab, zrows)

    out = _post(ptab, xpart, acc.reshape(_N, _YW), r8, wn2f, bn2f)
    return out.reshape(_N, _C, _PF)


def kernel(x_u, edge_index_u, We_u, be_u, Wn1_u, bn1_u, Wn2_u, bn2_u,
           x_v, edge_index_v, We_v, be_v, Wn1_v, bn1_v, Wn2_v, bn2_v,
           x_y, edge_index_y, We_y, be_y, Wn1_y, bn1_y, Wn2_y, bn2_y):
    ou = _plane_opt(x_u, edge_index_u, We_u, be_u, Wn1_u, bn1_u, Wn2_u, bn2_u)
    ov = _plane_opt(x_v, edge_index_v, We_v, be_v, Wn1_v, bn1_v, Wn2_v, bn2_v)
    oy = _plane_opt(x_y, edge_index_y, We_y, be_y, Wn1_y, bn1_y, Wn2_y, bn2_y)
    return (ou, ov, oy)
